# Initial kernel scaffold; baseline (speedup 1.0000x reference)
import jax
import jax.numpy as jnp
from jax.experimental import pallas as pl


def kernel(Y_user, Y_item, X_user, X_item, H_fwd, H_inv, w_fwd, w_inv, edge_src, edge_dst):
    raise NotImplementedError("write your pallas kernel here")


# speedup vs baseline: 4.5648x; 4.5648x over previous
import jax
import jax.numpy as jnp
from jax import lax
from jax.experimental import pallas as pl
from jax.experimental.pallas import tpu as pltpu
from jax.experimental.pallas import tpu_sc as plsc

N_USER = 10000
N_ITEM = 10000
E = 320000
K = 128
EPS = 0.1

NC = 2
NS = 16
L = 16

TILE_E = E // NS
CHUNK = 80
SUPC = 25
NSUP = TILE_E // (SUPC * CHUNK)
N_PAD = 10240
ROWS_PER_TILE = N_PAD // NS


def _sc_body(tab, gidx_h, sidx_h, w_h, zk,
             a_out, d_out,
             gidx_v, sidx_v, w_v, rows_v,
             accum_sh):
    core = lax.axis_index("c")
    tid = lax.axis_index("s")

    one16 = jnp.ones((L,), jnp.float32)

    tsl = pl.ds(tid * ROWS_PER_TILE, ROWS_PER_TILE)
    pltpu.sync_copy(zk, accum_sh.at[tsl])
    plsc.subcore_barrier()

    off = core * N_ITEM

    def super_step(s, scarry):
        pltpu.sync_copy(gidx_h.at[core, tid, s], gidx_v)
        pltpu.sync_copy(sidx_h.at[core, tid, s], sidx_v)
        pltpu.sync_copy(w_h.at[core, tid, s], w_v)

        def adj(r, carry):
            for k in range(CHUNK // L):
                sl = pl.ds(k * L, L)
                gidx_v[r, sl] = gidx_v[r, sl] + off
            return carry
        lax.fori_loop(0, SUPC, adj, 0)

        def chunk_step(c, carry):
            pltpu.sync_copy(tab.at[gidx_v.at[c]], rows_v)

            def kstep(k, kcarry):
                w_vec = w_v[c, pl.ds(k * L, L)]
                for li in range(L):
                    sel = jnp.full((L,), li, jnp.int32)
                    wb = jnp.take_along_axis(w_vec, sel, axis=0,
                                             mode="promise_in_bounds")
                    ri = k * L + li
                    for j in range(K // L):
                        sl = pl.ds(j * L, L)
                        rows_v[ri, sl] = rows_v[ri, sl] * wb
                return kcarry
            lax.fori_loop(0, CHUNK // L, kstep, 0)

            pltpu.sync_copy(rows_v, accum_sh.at[sidx_v.at[c]], add=True)
            return carry
        lax.fori_loop(0, SUPC, chunk_step, 0)
        return scarry
    lax.fori_loop(0, NSUP, super_step, 0)
    plsc.subcore_barrier()

    pltpu.sync_copy(accum_sh.at[tsl], a_out.at[core, tsl])
    pltpu.sync_copy(zk, accum_sh.at[tsl])

    def onesrow(i, carry):
        for j in range(K // L):
            rows_v[i, pl.ds(j * L, L)] = one16
        return carry
    lax.fori_loop(0, CHUNK, onesrow, 0)
    plsc.subcore_barrier()

    def dsup(s, scarry):
        pltpu.sync_copy(sidx_h.at[core, tid, s], sidx_v)

        def dchunk(c, carry):
            pltpu.sync_copy(rows_v, accum_sh.at[sidx_v.at[c]], add=True)
            return carry
        lax.fori_loop(0, SUPC, dchunk, 0)
        return scarry
    lax.fori_loop(0, NSUP, dsup, 0)
    plsc.subcore_barrier()
    pltpu.sync_copy(accum_sh.at[tsl], d_out.at[core, tsl])


def _sc_segment_sums(tab, gidx, sidx, wgt):
    out_type = (
        jax.ShapeDtypeStruct((NC, N_PAD, K), jnp.float32),
        jax.ShapeDtypeStruct((NC, N_PAD, K), jnp.float32),
    )
    mesh = plsc.VectorSubcoreMesh(
        core_axis_name="c", subcore_axis_name="s", num_cores=NC, num_subcores=NS)
    f = pl.kernel(
        _sc_body,
        out_type=out_type,
        mesh=mesh,
        scratch_types=[
            pltpu.VMEM((SUPC, CHUNK), jnp.int32),
            pltpu.VMEM((SUPC, CHUNK), jnp.int32),
            pltpu.VMEM((SUPC, CHUNK), jnp.float32),
            pltpu.VMEM((CHUNK, K), jnp.float32),
            pltpu.VMEM_SHARED((N_PAD, K), jnp.float32),
        ],
    )
    zk = jnp.zeros((ROWS_PER_TILE, K), jnp.float32)
    return f(tab, gidx, sidx, wgt, zk)


def _tc_body(a, d, xu, yu, xi, yi, hf, hi, ou, oi):
    scale = (EPS / K) * (EPS / K)
    msg_scale = EPS / K
    dn = (((1,), (1,)), ((), ()))
    hf_m = hf[...]
    hi_m = hi[...]
    hht_f = lax.dot_general(hf_m, hf_m, dn, preferred_element_type=jnp.float32)
    hht_i = lax.dot_general(hi_m, hi_m, dn, preferred_element_type=jnp.float32)
    echo_u = jnp.dot(yu[...], hht_f, preferred_element_type=jnp.float32)
    echo_i = jnp.dot(yi[...], hht_i, preferred_element_type=jnp.float32)
    msg_u = lax.dot_general(a[0], hf_m, dn, preferred_element_type=jnp.float32)
    msg_i = lax.dot_general(a[1], hi_m, dn, preferred_element_type=jnp.float32)
    ou[...] = xu[...] - d[0] * echo_u * scale + msg_u * msg_scale
    oi[...] = xi[...] - d[1] * echo_i * scale + msg_i * msg_scale


def _tc_combine(A, D, X_user, Y_user, X_item, Y_item, H_fwd, H_inv):
    BLK = 1000
    grid = (N_USER // BLK,)
    a_spec = pl.BlockSpec((NC, BLK, K), lambda b: (0, b, 0))
    d_spec = pl.BlockSpec((NC, BLK, 1), lambda b: (0, b, 0))
    row_spec = pl.BlockSpec((BLK, K), lambda b: (b, 0))
    h_spec = pl.BlockSpec((K, K), lambda b: (0, 0))
    return pl.pallas_call(
        _tc_body,
        grid=grid,
        in_specs=[a_spec, d_spec, row_spec, row_spec, row_spec, row_spec,
                  h_spec, h_spec],
        out_specs=[row_spec, row_spec],
        out_shape=[
            jax.ShapeDtypeStruct((N_USER, K), jnp.float32),
            jax.ShapeDtypeStruct((N_ITEM, K), jnp.float32),
        ],
    )(A, D, X_user, Y_user, X_item, Y_item, H_fwd, H_inv)


def kernel(Y_user, Y_item, X_user, X_item, H_fwd, H_inv, w_fwd, w_inv, edge_src, edge_dst):
    es_r = edge_src.reshape(NS, NSUP, SUPC, CHUNK)
    ed_r = edge_dst.reshape(NS, NSUP, SUPC, CHUNK)
    wf_r = w_fwd.reshape(NS, NSUP, SUPC, CHUNK)
    wi_r = w_inv.reshape(NS, NSUP, SUPC, CHUNK)
    tab = jnp.concatenate([Y_item, Y_user], axis=0)
    gidx = jnp.stack([ed_r, es_r])
    sidx = jnp.stack([es_r, ed_r])
    wgt = jnp.stack([wi_r, wf_r])
    A, D = _sc_segment_sums(tab, gidx, sidx, wgt)
    D2 = D[:, :, :1]
    return _tc_combine(A, D2, X_user, Y_user, X_item, Y_item, H_fwd, H_inv)


# speedup vs baseline: 5.3446x; 1.1708x over previous
import jax
import jax.numpy as jnp
from jax import lax
from jax.experimental import pallas as pl
from jax.experimental.pallas import tpu as pltpu
from jax.experimental.pallas import tpu_sc as plsc

N_USER = 10000
N_ITEM = 10000
E = 320000
K = 128
EPS = 0.1

NC = 2
NS = 16
L = 16

TILE_E = E // NS
CHUNK = 80
SUPC = 25
NSUP = TILE_E // (SUPC * CHUNK)
N_PAD = 10240
ROWS_PER_TILE = N_PAD // NS


def _sc_body(tab, gidx_h, sidx_h, w_h, zk,
             a_out, d_out,
             gidx_v, sidx_v, w_v, rows_a, rows_b,
             accum_sh,
             sem_a, sem_b, sem_p):
    core = lax.axis_index("c")
    tid = lax.axis_index("s")

    one16 = jnp.ones((L,), jnp.float32)

    tsl = pl.ds(tid * ROWS_PER_TILE, ROWS_PER_TILE)
    pltpu.sync_copy(zk, accum_sh.at[tsl])
    plsc.subcore_barrier()

    def scale_rows(buf, c):
        def kstep(k, kcarry):
            w_vec = w_v[c, pl.ds(k * L, L)]
            for li in range(L):
                sel = jnp.full((L,), li, jnp.int32)
                wb = jnp.take_along_axis(w_vec, sel, axis=0,
                                         mode="promise_in_bounds")
                ri = k * L + li
                for j in range(K // L):
                    sl = pl.ds(j * L, L)
                    buf[ri, sl] = buf[ri, sl] * wb
            return kcarry
        lax.fori_loop(0, CHUNK // L, kstep, 0)

    def s_start(buf, sem, c):
        pltpu.async_copy(buf, accum_sh.at[sidx_v.at[c]], sem, add=True)

    def s_wait(buf, sem):
        pltpu.make_async_copy(buf, accum_sh.at[sidx_v.at[0]], sem).wait()

    def super_step(s, scarry):
        pltpu.sync_copy(gidx_h.at[core, tid, s], gidx_v)
        pltpu.sync_copy(sidx_h.at[core, tid, s], sidx_v)
        pltpu.sync_copy(w_h.at[core, tid, s], w_v)

        pltpu.sync_copy(tab.at[gidx_v.at[0]], rows_a)
        scale_rows(rows_a, 0)
        s_start(rows_a, sem_a, 0)

        def pair(i, carry):
            c1 = 2 * i + 1
            c2 = 2 * i + 2
            pltpu.sync_copy(tab.at[gidx_v.at[c1]], rows_b)
            scale_rows(rows_b, c1)
            s_wait(rows_a, sem_a)
            s_start(rows_b, sem_b, c1)
            pltpu.sync_copy(tab.at[gidx_v.at[c2]], rows_a)
            scale_rows(rows_a, c2)
            s_wait(rows_b, sem_b)
            s_start(rows_a, sem_a, c2)
            return carry
        lax.fori_loop(0, (SUPC - 1) // 2, pair, 0)
        s_wait(rows_a, sem_a)
        return scarry
    lax.fori_loop(0, NSUP, super_step, 0)
    plsc.subcore_barrier()

    pltpu.sync_copy(accum_sh.at[tsl], a_out.at[core, tsl])
    pltpu.sync_copy(zk, accum_sh.at[tsl])

    def onesrow(i, carry):
        for j in range(K // L):
            rows_a[i, pl.ds(j * L, L)] = one16
        return carry
    lax.fori_loop(0, CHUNK, onesrow, 0)
    plsc.subcore_barrier()

    DK = 5

    def dsup(s, scarry):
        pltpu.sync_copy(sidx_h.at[core, tid, s], sidx_v)

        def dgroup(g, carry):
            def fire(c, fcarry):
                pltpu.async_copy(rows_a, accum_sh.at[sidx_v.at[g * DK + c]],
                                 sem_p, add=True)
                return fcarry
            lax.fori_loop(0, DK, fire, 0)

            def drain(c, dcarry):
                pltpu.make_async_copy(rows_a, accum_sh.at[sidx_v.at[0]],
                                      sem_p).wait()
                return dcarry
            lax.fori_loop(0, DK, drain, 0)
            return carry
        lax.fori_loop(0, SUPC // DK, dgroup, 0)
        return scarry
    lax.fori_loop(0, NSUP, dsup, 0)
    plsc.subcore_barrier()
    pltpu.sync_copy(accum_sh.at[tsl], d_out.at[core, tsl])


def _sc_segment_sums(tab, gidx, sidx, wgt):
    out_type = (
        jax.ShapeDtypeStruct((NC, N_PAD, K), jnp.float32),
        jax.ShapeDtypeStruct((NC, N_PAD, K), jnp.float32),
    )
    mesh = plsc.VectorSubcoreMesh(
        core_axis_name="c", subcore_axis_name="s", num_cores=NC, num_subcores=NS)
    f = pl.kernel(
        _sc_body,
        out_type=out_type,
        mesh=mesh,
        scratch_types=[
            pltpu.VMEM((SUPC, CHUNK), jnp.int32),
            pltpu.VMEM((SUPC, CHUNK), jnp.int32),
            pltpu.VMEM((SUPC, CHUNK), jnp.float32),
            pltpu.VMEM((CHUNK, K), jnp.float32),
            pltpu.VMEM((CHUNK, K), jnp.float32),
            pltpu.VMEM_SHARED((N_PAD, K), jnp.float32),
            pltpu.SemaphoreType.DMA,
            pltpu.SemaphoreType.DMA,
            pltpu.SemaphoreType.DMA,
        ],
    )
    zk = jnp.zeros((ROWS_PER_TILE, K), jnp.float32)
    return f(tab, gidx, sidx, wgt, zk)


def _tc_body(a, d, xu, yu, xi, yi, hf, hi, ou, oi):
    scale = (EPS / K) * (EPS / K)
    msg_scale = EPS / K
    dn = (((1,), (1,)), ((), ()))
    hf_m = hf[...]
    hi_m = hi[...]
    hht_f = lax.dot_general(hf_m, hf_m, dn, preferred_element_type=jnp.float32)
    hht_i = lax.dot_general(hi_m, hi_m, dn, preferred_element_type=jnp.float32)
    echo_u = jnp.dot(yu[...], hht_f, preferred_element_type=jnp.float32)
    echo_i = jnp.dot(yi[...], hht_i, preferred_element_type=jnp.float32)
    msg_u = lax.dot_general(a[0], hf_m, dn, preferred_element_type=jnp.float32)
    msg_i = lax.dot_general(a[1], hi_m, dn, preferred_element_type=jnp.float32)
    ou[...] = xu[...] - d[0] * echo_u * scale + msg_u * msg_scale
    oi[...] = xi[...] - d[1] * echo_i * scale + msg_i * msg_scale


def _tc_combine(A, D, X_user, Y_user, X_item, Y_item, H_fwd, H_inv):
    BLK = 1000
    grid = (N_USER // BLK,)
    a_spec = pl.BlockSpec((NC, BLK, K), lambda b: (0, b, 0))
    d_spec = pl.BlockSpec((NC, BLK, 1), lambda b: (0, b, 0))
    row_spec = pl.BlockSpec((BLK, K), lambda b: (b, 0))
    h_spec = pl.BlockSpec((K, K), lambda b: (0, 0))
    return pl.pallas_call(
        _tc_body,
        grid=grid,
        in_specs=[a_spec, d_spec, row_spec, row_spec, row_spec, row_spec,
                  h_spec, h_spec],
        out_specs=[row_spec, row_spec],
        out_shape=[
            jax.ShapeDtypeStruct((N_USER, K), jnp.float32),
            jax.ShapeDtypeStruct((N_ITEM, K), jnp.float32),
        ],
    )(A, D, X_user, Y_user, X_item, Y_item, H_fwd, H_inv)


def kernel(Y_user, Y_item, X_user, X_item, H_fwd, H_inv, w_fwd, w_inv, edge_src, edge_dst):
    es_r = edge_src.reshape(NS, NSUP, SUPC, CHUNK)
    ed_r = edge_dst.reshape(NS, NSUP, SUPC, CHUNK)
    wf_r = w_fwd.reshape(NS, NSUP, SUPC, CHUNK)
    wi_r = w_inv.reshape(NS, NSUP, SUPC, CHUNK)
    tab = jnp.concatenate([Y_item, Y_user], axis=0)
    gidx = jnp.stack([ed_r, es_r + N_ITEM])
    sidx = jnp.stack([es_r, ed_r])
    wgt = jnp.stack([wi_r, wf_r])
    A, D = _sc_segment_sums(tab, gidx, sidx, wgt)
    D2 = D[:, :, :1]
    return _tc_combine(A, D2, X_user, Y_user, X_item, Y_item, H_fwd, H_inv)


# speedup vs baseline: 7.0514x; 1.3194x over previous
import jax
import jax.numpy as jnp
from jax import lax
from jax.experimental import pallas as pl
from jax.experimental.pallas import tpu as pltpu
from jax.experimental.pallas import tpu_sc as plsc

N_USER = 10000
N_ITEM = 10000
E = 320000
K = 128
EPS = 0.1

NC = 2
NS = 16
L = 16

TILE_E = E // NS
CHUNK = 80
SUPC = 25
NSUP = TILE_E // (SUPC * CHUNK)
N_PAD = 10240
ROWS_PER_TILE = N_PAD // NS


def _sc_body(tab, gidx_h, sidx_h, w_h, zk,
             a_out, d_out,
             gidx_v, sidx_v, w_v, rows_a, rows_b, rows_c,
             accum_sh,
             sem_a, sem_b, sem_c, gsem_a, gsem_b, gsem_c, sem_p):
    core = lax.axis_index("c")
    tid = lax.axis_index("s")

    one16 = jnp.ones((L,), jnp.float32)

    tsl = pl.ds(tid * ROWS_PER_TILE, ROWS_PER_TILE)
    pltpu.sync_copy(zk, accum_sh.at[tsl])
    plsc.subcore_barrier()

    def scale_rows(buf, c):
        def kstep(k, kcarry):
            w_vec = w_v[c, pl.ds(k * L, L)]
            for li in range(L):
                sel = jnp.full((L,), li, jnp.int32)
                wb = jnp.take_along_axis(w_vec, sel, axis=0,
                                         mode="promise_in_bounds")
                ri = k * L + li
                for j in range(K // L):
                    sl = pl.ds(j * L, L)
                    buf[ri, sl] = buf[ri, sl] * wb
            return kcarry
        lax.fori_loop(0, CHUNK // L, kstep, 0)

    def s_start(buf, sem, c):
        pltpu.async_copy(buf, accum_sh.at[sidx_v.at[c]], sem, add=True)

    def s_wait(buf, sem):
        pltpu.make_async_copy(buf, accum_sh.at[sidx_v.at[0]], sem).wait()

    def g_start(buf, sem, c):
        pltpu.async_copy(tab.at[gidx_v.at[c]], buf, sem)

    def g_wait(buf, sem):
        pltpu.make_async_copy(tab.at[gidx_v.at[0]], buf, sem).wait()

    def super_step(s, scarry):
        pltpu.sync_copy(gidx_h.at[core, tid, s], gidx_v)
        pltpu.sync_copy(sidx_h.at[core, tid, s], sidx_v)
        pltpu.sync_copy(w_h.at[core, tid, s], w_v)

        g_start(rows_a, gsem_a, 0)
        g_start(rows_b, gsem_b, 1)
        g_wait(rows_a, gsem_a)
        scale_rows(rows_a, 0)
        s_start(rows_a, sem_a, 0)
        g_start(rows_c, gsem_c, 2)
        g_wait(rows_b, gsem_b)
        scale_rows(rows_b, 1)
        s_start(rows_b, sem_b, 1)
        s_wait(rows_a, sem_a)
        g_start(rows_a, gsem_a, 3)
        g_wait(rows_c, gsem_c)
        scale_rows(rows_c, 2)
        s_start(rows_c, sem_c, 2)

        def triple(i, carry):
            c0 = 3 * i
            s_wait(rows_b, sem_b)
            g_start(rows_b, gsem_b, c0 + 1)
            g_wait(rows_a, gsem_a)
            scale_rows(rows_a, c0)
            s_start(rows_a, sem_a, c0)
            s_wait(rows_c, sem_c)
            g_start(rows_c, gsem_c, c0 + 2)
            g_wait(rows_b, gsem_b)
            scale_rows(rows_b, c0 + 1)
            s_start(rows_b, sem_b, c0 + 1)
            s_wait(rows_a, sem_a)
            g_start(rows_a, gsem_a, c0 + 3)
            g_wait(rows_c, gsem_c)
            scale_rows(rows_c, c0 + 2)
            s_start(rows_c, sem_c, c0 + 2)
            return carry
        lax.fori_loop(1, (SUPC - 1) // 3, triple, 0)

        g_wait(rows_a, gsem_a)
        scale_rows(rows_a, SUPC - 1)
        s_start(rows_a, sem_a, SUPC - 1)
        s_wait(rows_b, sem_b)
        s_wait(rows_c, sem_c)
        s_wait(rows_a, sem_a)
        return scarry
    lax.fori_loop(0, NSUP, super_step, 0)
    plsc.subcore_barrier()

    pltpu.sync_copy(accum_sh.at[tsl], a_out.at[core, tsl])
    pltpu.sync_copy(zk, accum_sh.at[tsl])

    def onesrow(i, carry):
        for j in range(K // L):
            rows_a[i, pl.ds(j * L, L)] = one16
        return carry
    lax.fori_loop(0, CHUNK, onesrow, 0)
    plsc.subcore_barrier()

    DK = 5

    def dsup(s, scarry):
        pltpu.sync_copy(sidx_h.at[core, tid, s], sidx_v)

        def dgroup(g, carry):
            def fire(c, fcarry):
                pltpu.async_copy(rows_a, accum_sh.at[sidx_v.at[g * DK + c]],
                                 sem_p, add=True)
                return fcarry
            lax.fori_loop(0, DK, fire, 0)

            def drain(c, dcarry):
                pltpu.make_async_copy(rows_a, accum_sh.at[sidx_v.at[0]],
                                      sem_p).wait()
                return dcarry
            lax.fori_loop(0, DK, drain, 0)
            return carry
        lax.fori_loop(0, SUPC // DK, dgroup, 0)
        return scarry
    lax.fori_loop(0, NSUP, dsup, 0)
    plsc.subcore_barrier()
    pltpu.sync_copy(accum_sh.at[tsl], d_out.at[core, tsl])


def _sc_segment_sums(tab, gidx, sidx, wgt):
    out_type = (
        jax.ShapeDtypeStruct((NC, N_PAD, K), jnp.float32),
        jax.ShapeDtypeStruct((NC, N_PAD, K), jnp.float32),
    )
    mesh = plsc.VectorSubcoreMesh(
        core_axis_name="c", subcore_axis_name="s", num_cores=NC, num_subcores=NS)
    f = pl.kernel(
        _sc_body,
        out_type=out_type,
        mesh=mesh,
        scratch_types=[
            pltpu.VMEM((SUPC, CHUNK), jnp.int32),
            pltpu.VMEM((SUPC, CHUNK), jnp.int32),
            pltpu.VMEM((SUPC, CHUNK), jnp.float32),
            pltpu.VMEM((CHUNK, K), jnp.float32),
            pltpu.VMEM((CHUNK, K), jnp.float32),
            pltpu.VMEM((CHUNK, K), jnp.float32),
            pltpu.VMEM_SHARED((N_PAD, K), jnp.float32),
            pltpu.SemaphoreType.DMA,
            pltpu.SemaphoreType.DMA,
            pltpu.SemaphoreType.DMA,
            pltpu.SemaphoreType.DMA,
            pltpu.SemaphoreType.DMA,
            pltpu.SemaphoreType.DMA,
            pltpu.SemaphoreType.DMA,
        ],
    )
    zk = jnp.zeros((ROWS_PER_TILE, K), jnp.float32)
    return f(tab, gidx, sidx, wgt, zk)


def _tc_body(a, d, xu, yu, xi, yi, hf, hi, ou, oi):
    scale = (EPS / K) * (EPS / K)
    msg_scale = EPS / K
    dn = (((1,), (1,)), ((), ()))
    hf_m = hf[...]
    hi_m = hi[...]
    hht_f = lax.dot_general(hf_m, hf_m, dn, preferred_element_type=jnp.float32)
    hht_i = lax.dot_general(hi_m, hi_m, dn, preferred_element_type=jnp.float32)
    echo_u = jnp.dot(yu[...], hht_f, preferred_element_type=jnp.float32)
    echo_i = jnp.dot(yi[...], hht_i, preferred_element_type=jnp.float32)
    msg_u = lax.dot_general(a[0], hf_m, dn, preferred_element_type=jnp.float32)
    msg_i = lax.dot_general(a[1], hi_m, dn, preferred_element_type=jnp.float32)
    ou[...] = xu[...] - d[0] * echo_u * scale + msg_u * msg_scale
    oi[...] = xi[...] - d[1] * echo_i * scale + msg_i * msg_scale


def _tc_combine(A, D, X_user, Y_user, X_item, Y_item, H_fwd, H_inv):
    BLK = 1000
    grid = (N_USER // BLK,)
    a_spec = pl.BlockSpec((NC, BLK, K), lambda b: (0, b, 0))
    d_spec = pl.BlockSpec((NC, BLK, 1), lambda b: (0, b, 0))
    row_spec = pl.BlockSpec((BLK, K), lambda b: (b, 0))
    h_spec = pl.BlockSpec((K, K), lambda b: (0, 0))
    return pl.pallas_call(
        _tc_body,
        grid=grid,
        in_specs=[a_spec, d_spec, row_spec, row_spec, row_spec, row_spec,
                  h_spec, h_spec],
        out_specs=[row_spec, row_spec],
        out_shape=[
            jax.ShapeDtypeStruct((N_USER, K), jnp.float32),
            jax.ShapeDtypeStruct((N_ITEM, K), jnp.float32),
        ],
    )(A, D, X_user, Y_user, X_item, Y_item, H_fwd, H_inv)


def kernel(Y_user, Y_item, X_user, X_item, H_fwd, H_inv, w_fwd, w_inv, edge_src, edge_dst):
    es_r = edge_src.reshape(NS, NSUP, SUPC, CHUNK)
    ed_r = edge_dst.reshape(NS, NSUP, SUPC, CHUNK)
    wf_r = w_fwd.reshape(NS, NSUP, SUPC, CHUNK)
    wi_r = w_inv.reshape(NS, NSUP, SUPC, CHUNK)
    tab = jnp.concatenate([Y_item, Y_user], axis=0)
    gidx = jnp.stack([ed_r, es_r + N_ITEM])
    sidx = jnp.stack([es_r, ed_r])
    wgt = jnp.stack([wi_r, wf_r])
    A, D = _sc_segment_sums(tab, gidx, sidx, wgt)
    D2 = D[:, :, :1]
    return _tc_combine(A, D2, X_user, Y_user, X_item, Y_item, H_fwd, H_inv)


# speedup vs baseline: 7.0640x; 1.0018x over previous
import jax
import jax.numpy as jnp
from jax import lax
from jax.experimental import pallas as pl
from jax.experimental.pallas import tpu as pltpu
from jax.experimental.pallas import tpu_sc as plsc

N_USER = 10000
N_ITEM = 10000
E = 320000
K = 128
EPS = 0.1

NC = 2
NS = 16
L = 16

TILE_E = E // NS
CHUNK = 80
SUPC = 25
NSUP = TILE_E // (SUPC * CHUNK)
N_PAD = 10240
ROWS_PER_TILE = N_PAD // NS


def _sc_body(tab, gidx_h, sidx_h, w_h, zk,
             a_out, d_out,
             gidx_v, sidx_v, w_v, rows_a, rows_b, rows_c,
             accum_sh,
             sem_a, sem_b, sem_c, gsem_a, gsem_b, gsem_c, sem_p):
    core = lax.axis_index("c")
    tid = lax.axis_index("s")

    one16 = jnp.ones((L,), jnp.float32)

    tsl = pl.ds(tid * ROWS_PER_TILE, ROWS_PER_TILE)
    pltpu.sync_copy(zk, accum_sh.at[tsl])
    plsc.subcore_barrier()

    def scale_rows(buf, c):
        def kstep(k, kcarry):
            w_vec = w_v[c, pl.ds(k * L, L)]
            for li in range(L):
                sel = jnp.full((L,), li, jnp.int32)
                wb = jnp.take_along_axis(w_vec, sel, axis=0,
                                         mode="promise_in_bounds")
                ri = k * L + li
                for j in range(K // L):
                    sl = pl.ds(j * L, L)
                    buf[ri, sl] = buf[ri, sl] * wb
            return kcarry
        lax.fori_loop(0, CHUNK // L, kstep, 0)

    def s_start(buf, sem, c):
        pltpu.async_copy(buf, accum_sh.at[sidx_v.at[c]], sem, add=True)

    def s_wait(buf, sem):
        pltpu.make_async_copy(buf, accum_sh.at[sidx_v.at[0]], sem).wait()

    def g_start(buf, sem, c):
        pltpu.async_copy(tab.at[gidx_v.at[c]], buf, sem)

    def g_wait(buf, sem):
        pltpu.make_async_copy(tab.at[gidx_v.at[0]], buf, sem).wait()

    def super_step(s, scarry):
        pltpu.sync_copy(gidx_h.at[core, tid, s], gidx_v)
        pltpu.sync_copy(sidx_h.at[core, tid, s], sidx_v)
        pltpu.sync_copy(w_h.at[core, tid, s], w_v)

        g_start(rows_a, gsem_a, 0)
        g_start(rows_b, gsem_b, 1)
        g_wait(rows_a, gsem_a)
        scale_rows(rows_a, 0)
        s_start(rows_a, sem_a, 0)
        g_start(rows_c, gsem_c, 2)
        g_wait(rows_b, gsem_b)
        scale_rows(rows_b, 1)
        s_start(rows_b, sem_b, 1)
        s_wait(rows_a, sem_a)
        g_start(rows_a, gsem_a, 3)
        g_wait(rows_c, gsem_c)
        scale_rows(rows_c, 2)
        s_start(rows_c, sem_c, 2)

        def triple(i, carry):
            c0 = 3 * i
            s_wait(rows_b, sem_b)
            g_start(rows_b, gsem_b, c0 + 1)
            g_wait(rows_a, gsem_a)
            scale_rows(rows_a, c0)
            s_start(rows_a, sem_a, c0)
            s_wait(rows_c, sem_c)
            g_start(rows_c, gsem_c, c0 + 2)
            g_wait(rows_b, gsem_b)
            scale_rows(rows_b, c0 + 1)
            s_start(rows_b, sem_b, c0 + 1)
            s_wait(rows_a, sem_a)
            g_start(rows_a, gsem_a, c0 + 3)
            g_wait(rows_c, gsem_c)
            scale_rows(rows_c, c0 + 2)
            s_start(rows_c, sem_c, c0 + 2)
            return carry
        lax.fori_loop(1, (SUPC - 1) // 3, triple, 0)

        g_wait(rows_a, gsem_a)
        scale_rows(rows_a, SUPC - 1)
        s_start(rows_a, sem_a, SUPC - 1)
        s_wait(rows_b, sem_b)
        s_wait(rows_c, sem_c)
        s_wait(rows_a, sem_a)
        return scarry
    lax.fori_loop(0, NSUP, super_step, 0)
    plsc.subcore_barrier()

    pltpu.sync_copy(accum_sh.at[tsl], a_out.at[core, tsl])
    pltpu.sync_copy(zk, accum_sh.at[tsl])

    def onesrow(i, carry):
        for j in range(K // L):
            rows_a[i, pl.ds(j * L, L)] = one16
        return carry
    lax.fori_loop(0, CHUNK, onesrow, 0)
    plsc.subcore_barrier()

    DK = 25

    def dsup(s, scarry):
        pltpu.sync_copy(sidx_h.at[core, tid, s], sidx_v)

        def dgroup(g, carry):
            def fire(c, fcarry):
                pltpu.async_copy(rows_a, accum_sh.at[sidx_v.at[g * DK + c]],
                                 sem_p, add=True)
                return fcarry
            lax.fori_loop(0, DK, fire, 0)

            def drain(c, dcarry):
                pltpu.make_async_copy(rows_a, accum_sh.at[sidx_v.at[0]],
                                      sem_p).wait()
                return dcarry
            lax.fori_loop(0, DK, drain, 0)
            return carry
        lax.fori_loop(0, SUPC // DK, dgroup, 0)
        return scarry
    lax.fori_loop(0, NSUP, dsup, 0)
    plsc.subcore_barrier()
    pltpu.sync_copy(accum_sh.at[tsl], d_out.at[core, tsl])


def _sc_segment_sums(tab, gidx, sidx, wgt):
    out_type = (
        jax.ShapeDtypeStruct((NC, N_PAD, K), jnp.float32),
        jax.ShapeDtypeStruct((NC, N_PAD, K), jnp.float32),
    )
    mesh = plsc.VectorSubcoreMesh(
        core_axis_name="c", subcore_axis_name="s", num_cores=NC, num_subcores=NS)
    f = pl.kernel(
        _sc_body,
        out_type=out_type,
        mesh=mesh,
        scratch_types=[
            pltpu.VMEM((SUPC, CHUNK), jnp.int32),
            pltpu.VMEM((SUPC, CHUNK), jnp.int32),
            pltpu.VMEM((SUPC, CHUNK), jnp.float32),
            pltpu.VMEM((CHUNK, K), jnp.float32),
            pltpu.VMEM((CHUNK, K), jnp.float32),
            pltpu.VMEM((CHUNK, K), jnp.float32),
            pltpu.VMEM_SHARED((N_PAD, K), jnp.float32),
            pltpu.SemaphoreType.DMA,
            pltpu.SemaphoreType.DMA,
            pltpu.SemaphoreType.DMA,
            pltpu.SemaphoreType.DMA,
            pltpu.SemaphoreType.DMA,
            pltpu.SemaphoreType.DMA,
            pltpu.SemaphoreType.DMA,
        ],
    )
    zk = jnp.zeros((ROWS_PER_TILE, K), jnp.float32)
    return f(tab, gidx, sidx, wgt, zk)


def _tc_body(a, d, xu, yu, xi, yi, hf, hi, ou, oi):
    scale = (EPS / K) * (EPS / K)
    msg_scale = EPS / K
    dn = (((1,), (1,)), ((), ()))
    hf_m = hf[...]
    hi_m = hi[...]
    hht_f = lax.dot_general(hf_m, hf_m, dn, preferred_element_type=jnp.float32)
    hht_i = lax.dot_general(hi_m, hi_m, dn, preferred_element_type=jnp.float32)
    echo_u = jnp.dot(yu[...], hht_f, preferred_element_type=jnp.float32)
    echo_i = jnp.dot(yi[...], hht_i, preferred_element_type=jnp.float32)
    msg_u = lax.dot_general(a[0], hf_m, dn, preferred_element_type=jnp.float32)
    msg_i = lax.dot_general(a[1], hi_m, dn, preferred_element_type=jnp.float32)
    ou[...] = xu[...] - d[0] * echo_u * scale + msg_u * msg_scale
    oi[...] = xi[...] - d[1] * echo_i * scale + msg_i * msg_scale


def _tc_combine(A, D, X_user, Y_user, X_item, Y_item, H_fwd, H_inv):
    BLK = 1000
    grid = (N_USER // BLK,)
    a_spec = pl.BlockSpec((NC, BLK, K), lambda b: (0, b, 0))
    d_spec = pl.BlockSpec((NC, BLK, 1), lambda b: (0, b, 0))
    row_spec = pl.BlockSpec((BLK, K), lambda b: (b, 0))
    h_spec = pl.BlockSpec((K, K), lambda b: (0, 0))
    return pl.pallas_call(
        _tc_body,
        grid=grid,
        in_specs=[a_spec, d_spec, row_spec, row_spec, row_spec, row_spec,
                  h_spec, h_spec],
        out_specs=[row_spec, row_spec],
        out_shape=[
            jax.ShapeDtypeStruct((N_USER, K), jnp.float32),
            jax.ShapeDtypeStruct((N_ITEM, K), jnp.float32),
        ],
    )(A, D, X_user, Y_user, X_item, Y_item, H_fwd, H_inv)


def kernel(Y_user, Y_item, X_user, X_item, H_fwd, H_inv, w_fwd, w_inv, edge_src, edge_dst):
    es_r = edge_src.reshape(NS, NSUP, SUPC, CHUNK)
    ed_r = edge_dst.reshape(NS, NSUP, SUPC, CHUNK)
    wf_r = w_fwd.reshape(NS, NSUP, SUPC, CHUNK)
    wi_r = w_inv.reshape(NS, NSUP, SUPC, CHUNK)
    tab = jnp.concatenate([Y_item, Y_user], axis=0)
    gidx = jnp.stack([ed_r, es_r + N_ITEM])
    sidx = jnp.stack([es_r, ed_r])
    wgt = jnp.stack([wi_r, wf_r])
    A, D = _sc_segment_sums(tab, gidx, sidx, wgt)
    D2 = D[:, :, :1]
    return _tc_combine(A, D2, X_user, Y_user, X_item, Y_item, H_fwd, H_inv)
